# Initial kernel scaffold; baseline (speedup 1.0000x reference)
import jax
import jax.numpy as jnp
from jax.experimental import pallas as pl


def kernel(x, edge_index, W_self1, W_neigh1, b1, W_self2, W_neigh2, b2):
    raise NotImplementedError("write your pallas kernel here")


# speedup vs baseline: 7.7490x; 7.7490x over previous
import functools

import jax
import jax.numpy as jnp
from jax import lax
from jax.experimental import pallas as pl
from jax.experimental.pallas import tpu as pltpu
from jax.experimental.pallas import tpu_sc as plsc

N = 10000
E = 320000
D = 128

NC = 2
NS = 16
NW = NC * NS
CH = 80
EW = E // NW
KW = EW // CH
RS = N // NS

_mesh = plsc.VectorSubcoreMesh(
    core_axis_name="c", subcore_axis_name="s", num_cores=NC, num_subcores=NS
)


@functools.partial(
    pl.kernel,
    out_type=(
        jax.ShapeDtypeStruct((NC, N, D), jnp.float32),
        jax.ShapeDtypeStruct((NW, N), jnp.float32),
    ),
    mesh=_mesh,
    scratch_types=[
        pltpu.VMEM((KW, CH), jnp.int32),
        pltpu.VMEM((KW, CH), jnp.int32),
        pltpu.VMEM((CH, D), jnp.float32),
        pltpu.VMEM((N,), jnp.float32),
        pltpu.VMEM_SHARED((N, D), jnp.float32),
        pltpu.SemaphoreType.DMA,
    ],
    compiler_params=pltpu.CompilerParams(
        use_tc_tiling_on_sc=False, needs_layout_passes=False
    ),
)
def _sc_agg(table, srcs, dsts, zeros, out_acc, out_deg,
            src_v, dst_v, rows_v, deg_v, acc_sh, sem):
    c = lax.axis_index("c")
    s = lax.axis_index("s")
    wid = s * NC + c
    r0 = s * RS

    pltpu.sync_copy(zeros.at[pl.ds(r0, RS)], acc_sh.at[pl.ds(r0, RS)])
    pltpu.sync_copy(srcs.at[pl.ds(wid * KW, KW)], src_v)
    pltpu.sync_copy(dsts.at[pl.ds(wid * KW, KW)], dst_v)

    def zero_deg(i, carry):
        deg_v[pl.ds(i * 16, 16)] = jnp.zeros((16,), jnp.float32)
        return carry

    lax.fori_loop(0, N // 16, zero_deg, 0)
    plsc.subcore_barrier()

    ones16 = jnp.ones((16,), jnp.float32)

    def body(k, carry):
        pltpu.async_copy(table.at[src_v.at[k]], rows_v, sem).wait()
        pltpu.sync_copy(rows_v, acc_sh.at[dst_v.at[k]], add=True)

        def deg_upd(j, carry2):
            idx = dst_v[k, pl.ds(j * 16, 16)]
            plsc.addupdate_scatter(deg_v, [idx], ones16)
            return carry2

        lax.fori_loop(0, CH // 16, deg_upd, 0)
        return carry

    lax.fori_loop(0, KW, body, 0)
    plsc.subcore_barrier()

    pltpu.sync_copy(acc_sh.at[pl.ds(r0, RS)], out_acc.at[c].at[pl.ds(r0, RS)])
    pltpu.sync_copy(deg_v, out_deg.at[wid])


BN = 1000


def _combine_body(relu, h_ref, acc_ref, deg_ref, ws_ref, wn_ref, b_ref, o_ref):
    h = h_ref[...]
    agg = acc_ref[0] + acc_ref[1]
    inv = 1.0 / jnp.maximum(deg_ref[...], 1.0)
    hn = agg * inv
    o = (
        jnp.dot(h, ws_ref[...], preferred_element_type=jnp.float32)
        + jnp.dot(hn, wn_ref[...], preferred_element_type=jnp.float32)
        + b_ref[...]
    )
    if relu:
        o = jnp.maximum(o, 0.0)
    o_ref[...] = o


def _combine(h, acc, deg, ws, wn, b, relu):
    return pl.pallas_call(
        functools.partial(_combine_body, relu),
        grid=(N // BN,),
        in_specs=[
            pl.BlockSpec((BN, D), lambda i: (i, 0)),
            pl.BlockSpec((2, BN, D), lambda i: (0, i, 0)),
            pl.BlockSpec((BN, 1), lambda i: (i, 0)),
            pl.BlockSpec((D, D), lambda i: (0, 0)),
            pl.BlockSpec((D, D), lambda i: (0, 0)),
            pl.BlockSpec((1, D), lambda i: (0, 0)),
        ],
        out_specs=pl.BlockSpec((BN, D), lambda i: (i, 0)),
        out_shape=jax.ShapeDtypeStruct((N, D), jnp.float32),
    )(h, acc, deg, ws, wn, b)


def kernel(x, edge_index, W_self1, W_neigh1, b1, W_self2, W_neigh2, b2):
    src = edge_index[0].reshape(E // CH, CH)
    dst = edge_index[1].reshape(E // CH, CH)
    zeros = jnp.zeros((N, D), jnp.float32)

    acc1, deg_parts = _sc_agg(x, src, dst, zeros)
    deg = jnp.sum(deg_parts, axis=0)[:, None]
    h1 = _combine(x, acc1, deg, W_self1, W_neigh1, b1[None, :], relu=True)
    acc2, _ = _sc_agg(h1, src, dst, zeros)
    out = _combine(h1, acc2, deg, W_self2, W_neigh2, b2[None, :], relu=False)
    return out


# speedup vs baseline: 12.4221x; 1.6031x over previous
import functools

import jax
import jax.numpy as jnp
from jax import lax
from jax.experimental import pallas as pl
from jax.experimental.pallas import tpu as pltpu
from jax.experimental.pallas import tpu_sc as plsc

N = 10000
E = 320000
D = 128

NC = 2
NS = 16
NW = NC * NS
CH = 80
EW = E // NW
KW = EW // CH
RS = N // NS

_mesh = plsc.VectorSubcoreMesh(
    core_axis_name="c", subcore_axis_name="s", num_cores=NC, num_subcores=NS
)


def _make_sc_agg(want_deg):
    out_type = [jax.ShapeDtypeStruct((NC, N, D), jnp.float32)]
    scratch = [
        pltpu.VMEM((KW, CH), jnp.int32),
        pltpu.VMEM((KW, CH), jnp.int32),
        pltpu.VMEM((CH, D), jnp.float32),
        pltpu.VMEM((CH, D), jnp.float32),
        pltpu.VMEM_SHARED((N, D), jnp.float32),
        pltpu.SemaphoreType.DMA,
        pltpu.SemaphoreType.DMA,
    ]
    if want_deg:
        out_type.append(jax.ShapeDtypeStruct((NW, N), jnp.float32))
        scratch.append(pltpu.VMEM((N,), jnp.float32))

    @functools.partial(
        pl.kernel,
        out_type=tuple(out_type) if want_deg else out_type[0],
        mesh=_mesh,
        scratch_types=scratch,
        compiler_params=pltpu.CompilerParams(
            use_tc_tiling_on_sc=False, needs_layout_passes=False
        ),
    )
    def _sc_agg(table, srcs, dsts, zeros, *rest):
        if want_deg:
            out_acc, out_deg, src_v, dst_v, buf_a, buf_b, acc_sh, sem_a, sem_b, deg_v = rest
        else:
            out_acc, src_v, dst_v, buf_a, buf_b, acc_sh, sem_a, sem_b = rest
        c = lax.axis_index("c")
        s = lax.axis_index("s")
        wid = s * NC + c
        r0 = s * RS

        pltpu.sync_copy(zeros.at[pl.ds(r0, RS)], acc_sh.at[pl.ds(r0, RS)])
        pltpu.sync_copy(srcs.at[pl.ds(wid * KW, KW)], src_v)
        pltpu.sync_copy(dsts.at[pl.ds(wid * KW, KW)], dst_v)

        if want_deg:
            def zero_deg(i, carry):
                deg_v[pl.ds(i * 16, 16)] = jnp.zeros((16,), jnp.float32)
                return carry

            lax.fori_loop(0, N // 16, zero_deg, 0)
        plsc.subcore_barrier()

        ones16 = jnp.ones((16,), jnp.float32)

        def gather(k, buf, sem):
            pltpu.async_copy(table.at[src_v.at[k]], buf, sem)

        def finish(k, buf, sem):
            pltpu.make_async_copy(table.at[src_v.at[k]], buf, sem).wait()
            pltpu.sync_copy(buf, acc_sh.at[dst_v.at[k]], add=True)
            if want_deg:
                def deg_upd(j, carry2):
                    idx = dst_v[k, pl.ds(j * 16, 16)]
                    plsc.addupdate_scatter(deg_v, [idx], ones16)
                    return carry2

                lax.fori_loop(0, CH // 16, deg_upd, 0)

        gather(0, buf_a, sem_a)

        def body(g, carry):
            ka = 2 * g
            gather(ka + 1, buf_b, sem_b)
            finish(ka, buf_a, sem_a)

            @pl.when(ka + 2 < KW)
            def _():
                gather(ka + 2, buf_a, sem_a)

            finish(ka + 1, buf_b, sem_b)
            return carry

        lax.fori_loop(0, KW // 2, body, 0)
        if KW % 2:
            finish(KW - 1, buf_a, sem_a)
        plsc.subcore_barrier()

        pltpu.sync_copy(acc_sh.at[pl.ds(r0, RS)], out_acc.at[c].at[pl.ds(r0, RS)])
        if want_deg:
            pltpu.sync_copy(deg_v, out_deg.at[wid])

    return _sc_agg


_sc_agg_deg = _make_sc_agg(True)
_sc_agg_plain = _make_sc_agg(False)


BN = 1000


def _combine_body(relu, h_ref, acc_ref, deg_ref, ws_ref, wn_ref, b_ref, o_ref):
    h = h_ref[...]
    agg = acc_ref[0] + acc_ref[1]
    inv = 1.0 / jnp.maximum(deg_ref[...], 1.0)
    hn = agg * inv
    o = (
        jnp.dot(h, ws_ref[...], preferred_element_type=jnp.float32)
        + jnp.dot(hn, wn_ref[...], preferred_element_type=jnp.float32)
        + b_ref[...]
    )
    if relu:
        o = jnp.maximum(o, 0.0)
    o_ref[...] = o


def _combine(h, acc, deg, ws, wn, b, relu):
    return pl.pallas_call(
        functools.partial(_combine_body, relu),
        grid=(N // BN,),
        in_specs=[
            pl.BlockSpec((BN, D), lambda i: (i, 0)),
            pl.BlockSpec((2, BN, D), lambda i: (0, i, 0)),
            pl.BlockSpec((BN, 1), lambda i: (i, 0)),
            pl.BlockSpec((D, D), lambda i: (0, 0)),
            pl.BlockSpec((D, D), lambda i: (0, 0)),
            pl.BlockSpec((1, D), lambda i: (0, 0)),
        ],
        out_specs=pl.BlockSpec((BN, D), lambda i: (i, 0)),
        out_shape=jax.ShapeDtypeStruct((N, D), jnp.float32),
    )(h, acc, deg, ws, wn, b)


def kernel(x, edge_index, W_self1, W_neigh1, b1, W_self2, W_neigh2, b2):
    src = edge_index[0].reshape(E // CH, CH)
    dst = edge_index[1].reshape(E // CH, CH)
    zeros = jnp.zeros((N, D), jnp.float32)

    acc1, deg_parts = _sc_agg_deg(x, src, dst, zeros)
    deg = jnp.sum(deg_parts, axis=0)[:, None]
    h1 = _combine(x, acc1, deg, W_self1, W_neigh1, b1[None, :], relu=True)
    acc2 = _sc_agg_plain(h1, src, dst, zeros)
    out = _combine(h1, acc2, deg, W_self2, W_neigh2, b2[None, :], relu=False)
    return out


# speedup vs baseline: 12.9434x; 1.0420x over previous
import functools

import jax
import jax.numpy as jnp
from jax import lax
from jax.experimental import pallas as pl
from jax.experimental.pallas import tpu as pltpu
from jax.experimental.pallas import tpu_sc as plsc

N = 10000
E = 320000
D = 128

NC = 2
NS = 16
NW = NC * NS
CH = 100
EW = E // NW
KW = EW // CH
RS = N // NS

_mesh = plsc.VectorSubcoreMesh(
    core_axis_name="c", subcore_axis_name="s", num_cores=NC, num_subcores=NS
)


_SC_PARAMS = pltpu.CompilerParams(
    use_tc_tiling_on_sc=False, needs_layout_passes=False
)


@functools.partial(
    pl.kernel,
    out_type=jax.ShapeDtypeStruct((NC, N, D), jnp.float32),
    mesh=_mesh,
    scratch_types=[
        pltpu.VMEM((KW, CH), jnp.int32),
        pltpu.VMEM((KW, CH), jnp.int32),
        pltpu.VMEM((CH, D), jnp.float32),
        pltpu.VMEM((CH, D), jnp.float32),
        pltpu.VMEM_SHARED((N, D), jnp.float32),
        pltpu.SemaphoreType.DMA,
        pltpu.SemaphoreType.DMA,
    ],
    compiler_params=_SC_PARAMS,
)
def _sc_agg(table, srcs, dsts, zeros, out_acc,
            src_v, dst_v, buf_a, buf_b, acc_sh, sem_a, sem_b):
    c = lax.axis_index("c")
    s = lax.axis_index("s")
    wid = s * NC + c
    r0 = s * RS

    pltpu.sync_copy(zeros.at[pl.ds(r0, RS)], acc_sh.at[pl.ds(r0, RS)])
    pltpu.sync_copy(srcs.at[pl.ds(wid * KW, KW)], src_v)
    pltpu.sync_copy(dsts.at[pl.ds(wid * KW, KW)], dst_v)
    plsc.subcore_barrier()

    def gather(k, buf, sem):
        pltpu.async_copy(table.at[src_v.at[k]], buf, sem)

    def finish(k, buf, sem):
        pltpu.make_async_copy(table.at[src_v.at[k]], buf, sem).wait()
        pltpu.sync_copy(buf, acc_sh.at[dst_v.at[k]], add=True)

    gather(0, buf_a, sem_a)

    def body(g, carry):
        ka = 2 * g
        gather(ka + 1, buf_b, sem_b)
        finish(ka, buf_a, sem_a)

        @pl.when(ka + 2 < KW)
        def _():
            gather(ka + 2, buf_a, sem_a)

        finish(ka + 1, buf_b, sem_b)
        return carry

    lax.fori_loop(0, KW // 2, body, 0)
    plsc.subcore_barrier()

    pltpu.sync_copy(acc_sh.at[pl.ds(r0, RS)], out_acc.at[c].at[pl.ds(r0, RS)])


@functools.partial(
    pl.kernel,
    out_type=jax.ShapeDtypeStruct((NW, N), jnp.float32),
    mesh=_mesh,
    scratch_types=[
        pltpu.VMEM((EW,), jnp.int32),
        pltpu.VMEM((N,), jnp.float32),
    ],
    compiler_params=_SC_PARAMS,
)
def _sc_deg(dsts_flat, out_deg, dst_v, deg_v):
    c = lax.axis_index("c")
    s = lax.axis_index("s")
    wid = s * NC + c
    pltpu.sync_copy(dsts_flat.at[wid], dst_v)

    def zero_deg(i, carry):
        deg_v[pl.ds(i * 16, 16)] = jnp.zeros((16,), jnp.float32)
        return carry

    lax.fori_loop(0, N // 16, zero_deg, 0)
    ones16 = jnp.ones((16,), jnp.float32)

    def upd(j, carry):
        plsc.addupdate_scatter(deg_v, [dst_v[pl.ds(j * 16, 16)]], ones16)
        return carry

    lax.fori_loop(0, EW // 16, upd, 0)
    pltpu.sync_copy(deg_v, out_deg.at[wid])


BN = 1000


def _combine_body(relu, h_ref, acc_ref, deg_ref, ws_ref, wn_ref, b_ref, o_ref):
    h = h_ref[...]
    agg = acc_ref[0] + acc_ref[1]
    inv = 1.0 / jnp.maximum(deg_ref[...], 1.0)
    hn = agg * inv
    o = (
        jnp.dot(h, ws_ref[...], preferred_element_type=jnp.float32)
        + jnp.dot(hn, wn_ref[...], preferred_element_type=jnp.float32)
        + b_ref[...]
    )
    if relu:
        o = jnp.maximum(o, 0.0)
    o_ref[...] = o


def _combine(h, acc, deg, ws, wn, b, relu):
    return pl.pallas_call(
        functools.partial(_combine_body, relu),
        grid=(N // BN,),
        in_specs=[
            pl.BlockSpec((BN, D), lambda i: (i, 0)),
            pl.BlockSpec((2, BN, D), lambda i: (0, i, 0)),
            pl.BlockSpec((BN, 1), lambda i: (i, 0)),
            pl.BlockSpec((D, D), lambda i: (0, 0)),
            pl.BlockSpec((D, D), lambda i: (0, 0)),
            pl.BlockSpec((1, D), lambda i: (0, 0)),
        ],
        out_specs=pl.BlockSpec((BN, D), lambda i: (i, 0)),
        out_shape=jax.ShapeDtypeStruct((N, D), jnp.float32),
    )(h, acc, deg, ws, wn, b)


def kernel(x, edge_index, W_self1, W_neigh1, b1, W_self2, W_neigh2, b2):
    src = edge_index[0].reshape(E // CH, CH)
    dst = edge_index[1].reshape(E // CH, CH)
    zeros = jnp.zeros((N, D), jnp.float32)

    deg_parts = _sc_deg(edge_index[1].reshape(NW, EW))
    acc1 = _sc_agg(x, src, dst, zeros)
    deg = jnp.sum(deg_parts, axis=0)[:, None]
    h1 = _combine(x, acc1, deg, W_self1, W_neigh1, b1[None, :], relu=True)
    acc2 = _sc_agg(h1, src, dst, zeros)
    out = _combine(h1, acc2, deg, W_self2, W_neigh2, b2[None, :], relu=False)
    return out
